# baseline (speedup 1.0000x reference)
import jax
import jax.numpy as jnp
from jax.experimental import pallas as pl
from jax.experimental.pallas import tpu as pltpu

H_IN = W_IN = 28
C1, K1 = 10, 5
C2, K2 = 20, 3
H1 = H_IN - K1 + 1
HP = H1 // 2
H2 = HP - K2 + 1
NUM_CLASSES = 10
FC1_OUT = 500
NCPAD = 128
FC1_PAD = 512
FEAT = H2 * H2 * C2

BT = 1024
L1 = H1 * C1
L1P = 256
KC1 = K1 * W_IN
KC2 = K2 * L1P


def _fused_kernel(x_ref, w1_ref, b1_ref, w2_ref, b2_ref,
                  fw1_ref, fb1_ref, fw2_ref, fb2_ref, o_ref,
                  xb_ref, h1_ref, feat_ref):
    bt = o_ref.shape[0]
    xb_ref[...] = x_ref[...].astype(jnp.bfloat16)
    for g in range(6):
        o14 = jnp.dot(xb_ref[:, g * 4 * W_IN: g * 4 * W_IN + 8 * W_IN],
                      w1_ref[...],
                      preferred_element_type=jnp.float32)
        r = jnp.maximum(o14 + b1_ref[...], 0.0)
        for k in range(2):
            u = 2 * g + k
            t = jnp.maximum(r[:, (2 * k) * L1P: (2 * k) * L1P + L1],
                            r[:, (2 * k + 1) * L1P: (2 * k + 1) * L1P + L1])
            sh = jnp.concatenate([t[:, C1:], t[:, :C1]], axis=1)
            p = jnp.maximum(t, sh)
            h1_ref[:, u * L1P:(u + 1) * L1P] = jnp.concatenate(
                [p, p[:, :L1P - L1]], axis=1).astype(jnp.bfloat16)

    for y2 in range(H2):
        o2 = jnp.dot(h1_ref[:, y2 * L1P: y2 * L1P + KC2], w2_ref[...],
                     preferred_element_type=jnp.float32)
        a = jnp.maximum(o2 + b2_ref[...], 0.0)
        feat_ref[:, y2 * H2 * C2:(y2 + 1) * H2 * C2] = a.astype(jnp.bfloat16)

    hidden = jnp.maximum(
        jnp.dot(feat_ref[...], fw1_ref[...],
                preferred_element_type=jnp.float32) + fb1_ref[...], 0.0)
    logits = jnp.dot(hidden.astype(jnp.bfloat16), fw2_ref[...],
                     preferred_element_type=jnp.float32) + fb2_ref[...]
    lane = jax.lax.broadcasted_iota(jnp.int32, logits.shape, 1)
    masked = jnp.where(lane < NUM_CLASSES, logits, -1e30)
    m = jnp.max(masked, axis=-1, keepdims=True)
    s = jnp.sum(jnp.exp(masked - m), axis=-1, keepdims=True)
    o_ref[...] = (logits - (m + jnp.log(s)))[:, :NUM_CLASSES]


def _pack_banded(w1p, w2p):
    w1 = w1p[:, :C1].reshape(K1, K1, C1)
    xs = jnp.arange(H1)
    t1 = jnp.zeros((K1, W_IN, H1, C1), jnp.float32)
    for dx in range(K1):
        t1 = t1.at[:, xs + dx, xs, :].set(
            jnp.broadcast_to(w1[:, dx, None, :], (K1, H1, C1)))
    w1c = t1.reshape(KC1, L1)
    w1g = jnp.zeros((8 * W_IN, 4 * L1P), jnp.float32)
    for yy in range(4):
        w1g = w1g.at[yy * W_IN: yy * W_IN + KC1,
                     yy * L1P: yy * L1P + L1].set(w1c)

    w2 = w2p.reshape(C1, K2, K2, C2)
    x2s = jnp.arange(H2)
    t2 = jnp.zeros((K2, L1P, H2, C2), jnp.float32)
    for dx2 in range(K2):
        for c in range(C1):
            t2 = t2.at[:, 2 * C1 * (x2s + dx2) + c, x2s, :].set(
                jnp.broadcast_to(w2[c, :, dx2, None, :], (K2, H2, C2)))
    w2c = t2.reshape(KC2, H2 * C2)
    return w1g, w2c


def kernel(x_nchw, w1p, b1p, w2p, b2p, fw1p, fb1p, fw2p, fb2p):
    B = x_nchw.shape[0]
    assert B % BT == 0

    x2d = x_nchw.reshape(B, H_IN * W_IN)
    w1c, w2c = _pack_banded(w1p, w2p)
    w1c = w1c.astype(jnp.bfloat16)
    w2c = w2c.astype(jnp.bfloat16)
    fw1b = fw1p.astype(jnp.bfloat16)
    fw2b = fw2p.astype(jnp.bfloat16)
    b1row = jnp.zeros((1, 4 * L1P), jnp.float32)
    for yy in range(4):
        b1row = b1row.at[:, yy * L1P: yy * L1P + L1].set(
            jnp.tile(b1p[:, :C1], (1, H1)))
    b2row = jnp.tile(b2p, (1, H2))

    out = pl.pallas_call(
        _fused_kernel,
        out_shape=jax.ShapeDtypeStruct((B, NUM_CLASSES), jnp.float32),
        grid=(B // BT,),
        in_specs=[
                pl.BlockSpec((BT, H_IN * W_IN), lambda i: (i, 0)),
                pl.BlockSpec((8 * W_IN, 4 * L1P), lambda i: (0, 0)),
                pl.BlockSpec((1, 4 * L1P), lambda i: (0, 0)),
                pl.BlockSpec((KC2, H2 * C2), lambda i: (0, 0)),
                pl.BlockSpec((1, H2 * C2), lambda i: (0, 0)),
                pl.BlockSpec((FEAT, FC1_PAD), lambda i: (0, 0)),
                pl.BlockSpec((1, FC1_PAD), lambda i: (0, 0)),
                pl.BlockSpec((FC1_PAD, NCPAD), lambda i: (0, 0)),
                pl.BlockSpec((1, NCPAD), lambda i: (0, 0)),
        ],
        out_specs=pl.BlockSpec((BT, NUM_CLASSES), lambda i: (i, 0)),
        scratch_shapes=[
            pltpu.VMEM((BT, H_IN * W_IN), jnp.bfloat16),
            pltpu.VMEM((BT, HP * L1P), jnp.bfloat16),
            pltpu.VMEM((BT, FEAT), jnp.bfloat16),
        ],
        compiler_params=pltpu.CompilerParams(
            dimension_semantics=("arbitrary",),
            vmem_limit_bytes=48 * 1024 * 1024,
        ),
    )(x2d, w1c, b1row, w2c, b2row, fw1b, fb1p, fw2b, fb2p)

    return out


# speedup vs baseline: 1.0643x; 1.0643x over previous
import jax
import jax.numpy as jnp
from jax.experimental import pallas as pl
from jax.experimental.pallas import tpu as pltpu

H_IN = W_IN = 28
C1, K1 = 10, 5
C2, K2 = 20, 3
H1 = H_IN - K1 + 1
HP = H1 // 2
H2 = HP - K2 + 1
NUM_CLASSES = 10
FC1_OUT = 500
NCPAD = 128
FC1_PAD = 512
FEAT = H2 * H2 * C2

BT = 1024
L1 = H1 * C1
L1P = 256
KC1 = K1 * W_IN
KC2 = K2 * L1P


def _fused_kernel(x_ref, w1_ref, b1_ref, w2_ref, b2_ref,
                  fw1_ref, fb1_ref, fw2_ref, fb2_ref, o_ref,
                  xb_ref, h1_ref, feat_ref):
    bt = o_ref.shape[0]
    xb_ref[...] = x_ref[...].astype(jnp.bfloat16)
    for u in range(HP):
        rows = []
        for py in range(2):
            y = 2 * u + py
            o1 = jnp.dot(xb_ref[:, y * W_IN: y * W_IN + KC1], w1_ref[...],
                         preferred_element_type=jnp.float32)
            rows.append(jnp.maximum(o1 + b1_ref[...], 0.0))
        t = jnp.maximum(rows[0], rows[1])
        sh = jnp.concatenate([t[:, C1:], t[:, :C1]], axis=1)
        p = jnp.maximum(t, sh)
        h1_ref[:, u * L1P:(u + 1) * L1P] = jnp.concatenate(
            [p, p[:, :L1P - L1]], axis=1).astype(jnp.bfloat16)

    for y2 in range(H2):
        o2 = jnp.dot(h1_ref[:, y2 * L1P: y2 * L1P + KC2], w2_ref[...],
                     preferred_element_type=jnp.float32)
        a = jnp.maximum(o2 + b2_ref[...], 0.0)
        feat_ref[:, y2 * H2 * C2:(y2 + 1) * H2 * C2] = a.astype(jnp.bfloat16)

    hidden = jnp.maximum(
        jnp.dot(feat_ref[...], fw1_ref[...],
                preferred_element_type=jnp.float32) + fb1_ref[...], 0.0)
    logits = jnp.dot(hidden.astype(jnp.bfloat16), fw2_ref[...],
                     preferred_element_type=jnp.float32) + fb2_ref[...]
    lane = jax.lax.broadcasted_iota(jnp.int32, logits.shape, 1)
    masked = jnp.where(lane < NUM_CLASSES, logits, -1e30)
    m = jnp.max(masked, axis=-1, keepdims=True)
    s = jnp.sum(jnp.exp(masked - m), axis=-1, keepdims=True)
    o_ref[...] = (logits - (m + jnp.log(s)))[:, :NUM_CLASSES]


def _pack_banded(w1p, w2p):
    w1 = w1p[:, :C1].reshape(K1, K1, C1)
    xs = jnp.arange(H1)
    t1 = jnp.zeros((K1, W_IN, H1, C1), jnp.float32)
    for dx in range(K1):
        t1 = t1.at[:, xs + dx, xs, :].set(
            jnp.broadcast_to(w1[:, dx, None, :], (K1, H1, C1)))
    w1c = t1.reshape(KC1, L1)

    w2 = w2p.reshape(C1, K2, K2, C2)
    x2s = jnp.arange(H2)
    t2 = jnp.zeros((K2, L1P, H2, C2), jnp.float32)
    for dx2 in range(K2):
        for c in range(C1):
            t2 = t2.at[:, 2 * C1 * (x2s + dx2) + c, x2s, :].set(
                jnp.broadcast_to(w2[c, :, dx2, None, :], (K2, H2, C2)))
    w2c = t2.reshape(KC2, H2 * C2)
    return w1c, w2c


def kernel(x_nchw, w1p, b1p, w2p, b2p, fw1p, fb1p, fw2p, fb2p):
    B = x_nchw.shape[0]
    assert B % BT == 0

    x2d = x_nchw.reshape(B, H_IN * W_IN)
    w1c, w2c = _pack_banded(w1p, w2p)
    w1c = w1c.astype(jnp.bfloat16)
    w2c = w2c.astype(jnp.bfloat16)
    fw1b = fw1p.astype(jnp.bfloat16)
    fw2b = fw2p.astype(jnp.bfloat16)
    b1row = jnp.tile(b1p[:, :C1], (1, H1))
    b2row = jnp.tile(b2p, (1, H2))

    out = pl.pallas_call(
        _fused_kernel,
        out_shape=jax.ShapeDtypeStruct((B, NUM_CLASSES), jnp.float32),
        grid=(B // BT,),
        in_specs=[
                pl.BlockSpec((BT, H_IN * W_IN), lambda i: (i, 0)),
                pl.BlockSpec((KC1, L1), lambda i: (0, 0)),
                pl.BlockSpec((1, L1), lambda i: (0, 0)),
                pl.BlockSpec((KC2, H2 * C2), lambda i: (0, 0)),
                pl.BlockSpec((1, H2 * C2), lambda i: (0, 0)),
                pl.BlockSpec((FEAT, FC1_PAD), lambda i: (0, 0)),
                pl.BlockSpec((1, FC1_PAD), lambda i: (0, 0)),
                pl.BlockSpec((FC1_PAD, NCPAD), lambda i: (0, 0)),
                pl.BlockSpec((1, NCPAD), lambda i: (0, 0)),
        ],
        out_specs=pl.BlockSpec((BT, NUM_CLASSES), lambda i: (i, 0)),
        scratch_shapes=[
            pltpu.VMEM((BT, H_IN * W_IN), jnp.bfloat16),
            pltpu.VMEM((BT, HP * L1P), jnp.bfloat16),
            pltpu.VMEM((BT, FEAT), jnp.bfloat16),
        ],
        compiler_params=pltpu.CompilerParams(
            dimension_semantics=("arbitrary",),
            vmem_limit_bytes=48 * 1024 * 1024,
        ),
    )(x2d, w1c, b1row, w2c, b2row, fw1b, fb1p, fw2b, fb2p)

    return out
